# baseline (speedup 1.0000x reference)
import functools

import jax
import jax.numpy as jnp
from jax import lax
from jax.experimental import pallas as pl
from jax.experimental.pallas import tpu as pltpu
from jax.experimental.pallas import tpu_sc as plsc

B = 16
L = 2048
TOTAL = 16384
V = 8192
D = 128
NULL_ID = 0
START_ID = 2
END_ID = 3

NC = 2
NS = 16
LANES = 16
NW = NC * NS
PW = (B * L) // NW
FETCH = PW + 8
CHUNK = 128
NCH = PW // CHUNK
NBUF = 3


def _body(flat_hbm, cu_hbm, vocab_hbm, out_hbm,
          cu_v, tok_v, ids_v, rows_v, vocab_sp, gsem, ssem, stsem):
    sid = lax.axis_index("s")
    wid = lax.axis_index("c") * NS + sid
    b = wid // 2
    j0 = (wid % 2) * PW

    vshard = V // NS
    stage = pltpu.async_copy(vocab_hbm.at[pl.ds(sid * vshard, vshard)],
                             vocab_sp.at[pl.ds(sid * vshard, vshard)], stsem)

    pltpu.sync_copy(cu_hbm.at[pl.ds(0, 16)], cu_v)
    bvec = jnp.full((LANES,), b, jnp.int32)
    start = plsc.load_gather(cu_v, [bvec])[0]
    nxt = plsc.load_gather(cu_v, [jnp.minimum(bvec + 1, 15)])[0]
    seglen = jnp.where(b == B - 1, jnp.int32(TOTAL), nxt) - start

    s = start + j0 - 1
    sa = pl.multiple_of(jnp.maximum(0, jnp.minimum(s & ~7, TOTAL - FETCH)), 8)
    pltpu.sync_copy(flat_hbm.at[pl.ds(sa, FETCH)], tok_v)

    iota = jnp.arange(LANES, dtype=jnp.int32)

    def id_chunk(c, carry):
        for u in range(CHUNK // LANES):
            j = j0 + c * CHUNK + u * LANES + iota
            g = jnp.clip(start + j - 1, 0, TOTAL - 1)
            tok = plsc.load_gather(tok_v, [g - sa])
            pid = jnp.where(
                j == 0, jnp.int32(START_ID),
                jnp.where(j <= seglen, tok,
                          jnp.where(j == seglen + 1, jnp.int32(END_ID),
                                    jnp.int32(NULL_ID))))
            ids_v[c, pl.ds(u * LANES, LANES)] = pid
        return carry

    lax.fori_loop(0, NCH, id_chunk, 0)

    stage.wait()
    plsc.subcore_barrier()

    base = wid * PW
    gd = [None] * NBUF
    sd = [None] * NBUF
    for c in range(min(NBUF, NCH)):
        gd[c] = pltpu.async_copy(vocab_sp.at[ids_v.at[c]], rows_v.at[c], gsem)
    for c in range(NCH):
        p = c % NBUF
        gd[p].wait()
        sd[p] = pltpu.async_copy(
            rows_v.at[p], out_hbm.at[pl.ds(base + c * CHUNK, CHUNK)], ssem)
        nc = c + NBUF
        if nc < NCH:
            sd[p].wait()
            gd[p] = pltpu.async_copy(
                vocab_sp.at[ids_v.at[nc]], rows_v.at[p], gsem)
    for c in range(NCH - NBUF, NCH):
        sd[c % NBUF].wait()


_tokenize = functools.partial(
    pl.kernel,
    out_type=jax.ShapeDtypeStruct((B * L, D), jnp.float32),
    mesh=plsc.VectorSubcoreMesh(core_axis_name="c", subcore_axis_name="s"),
    compiler_params=pltpu.CompilerParams(needs_layout_passes=False),
    scratch_types=[
        pltpu.VMEM((16,), jnp.int32),
        pltpu.VMEM((FETCH,), jnp.int32),
        pltpu.VMEM((NCH, CHUNK), jnp.int32),
        pltpu.VMEM((NBUF, CHUNK, D), jnp.float32),
        pltpu.VMEM_SHARED((V, D), jnp.float32),
        pltpu.SemaphoreType.DMA,
        pltpu.SemaphoreType.DMA,
        pltpu.SemaphoreType.DMA,
    ],
)(_body)


@jax.jit
def kernel(flat_tokens, cu_seqlens, vocab_emb):
    out = _tokenize(flat_tokens, cu_seqlens.astype(jnp.int32), vocab_emb)
    return out.reshape(B, L, D)


# speedup vs baseline: 1.0949x; 1.0949x over previous
import functools

import jax
import jax.numpy as jnp
from jax import lax
from jax.experimental import pallas as pl
from jax.experimental.pallas import tpu as pltpu
from jax.experimental.pallas import tpu_sc as plsc

B = 16
L = 2048
TOTAL = 16384
V = 8192
D = 128
NULL_ID = 0
START_ID = 2
END_ID = 3

NC = 2
NS = 16
LANES = 16
NW = NC * NS
PW = (B * L) // NW
FETCH = PW + 8
CHUNK = 64
NCH = PW // CHUNK
NBUF = 4


def _body(flat_hbm, cu_hbm, vocab_hbm, out_hbm,
          cu_v, tok_v, ids_v, rows_v, null_v, vocab_sp, gsem, ssem, stsem):
    sid = lax.axis_index("s")
    wid = lax.axis_index("c") * NS + sid
    b = wid // 2
    j0 = (wid % 2) * PW

    vshard = V // NS
    stage = pltpu.async_copy(vocab_hbm.at[pl.ds(sid * vshard, vshard)],
                             vocab_sp.at[pl.ds(sid * vshard, vshard)], stsem)

    pltpu.sync_copy(vocab_hbm.at[pl.ds(0, 1)], null_v)
    pltpu.sync_copy(null_v, vocab_sp.at[pl.ds(V + sid, 1)])

    pltpu.sync_copy(cu_hbm.at[pl.ds(0, 16)], cu_v)
    bvec = jnp.full((LANES,), b, jnp.int32)
    start = plsc.load_gather(cu_v, [bvec])[0]
    nxt = plsc.load_gather(cu_v, [jnp.minimum(bvec + 1, 15)])[0]
    seglen = jnp.where(b == B - 1, jnp.int32(TOTAL), nxt) - start

    s = start + j0 - 1
    sa = pl.multiple_of(jnp.maximum(0, jnp.minimum(s & ~7, TOTAL - FETCH)), 8)
    pltpu.sync_copy(flat_hbm.at[pl.ds(sa, FETCH)], tok_v)

    iota = jnp.arange(LANES, dtype=jnp.int32)

    def id_chunk(c, carry):
        for u in range(CHUNK // LANES):
            j = j0 + c * CHUNK + u * LANES + iota
            g = jnp.clip(start + j - 1, 0, TOTAL - 1)
            tok = plsc.load_gather(tok_v, [g - sa])
            pid = jnp.where(
                j == 0, jnp.int32(START_ID),
                jnp.where(j <= seglen, tok,
                          jnp.where(j == seglen + 1, jnp.int32(END_ID),
                                    jnp.int32(V) + sid)))
            ids_v[c, pl.ds(u * LANES, LANES)] = pid
        return carry

    lax.fori_loop(0, NCH, id_chunk, 0)

    stage.wait()
    plsc.subcore_barrier()

    base = wid * PW
    gd = [None] * NBUF
    sd = [None] * NBUF
    for c in range(min(NBUF, NCH)):
        gd[c] = pltpu.async_copy(vocab_sp.at[ids_v.at[c]], rows_v.at[c], gsem)
    for c in range(NCH):
        p = c % NBUF
        gd[p].wait()
        sd[p] = pltpu.async_copy(
            rows_v.at[p], out_hbm.at[pl.ds(base + c * CHUNK, CHUNK)], ssem)
        nc = c + NBUF
        if nc < NCH:
            sd[p].wait()
            gd[p] = pltpu.async_copy(
                vocab_sp.at[ids_v.at[nc]], rows_v.at[p], gsem)
    for c in range(NCH - NBUF, NCH):
        sd[c % NBUF].wait()


_tokenize = functools.partial(
    pl.kernel,
    out_type=jax.ShapeDtypeStruct((B * L, D), jnp.float32),
    mesh=plsc.VectorSubcoreMesh(core_axis_name="c", subcore_axis_name="s"),
    compiler_params=pltpu.CompilerParams(needs_layout_passes=False),
    scratch_types=[
        pltpu.VMEM((16,), jnp.int32),
        pltpu.VMEM((FETCH,), jnp.int32),
        pltpu.VMEM((NCH, CHUNK), jnp.int32),
        pltpu.VMEM((NBUF, CHUNK, D), jnp.float32),
        pltpu.VMEM((1, D), jnp.float32),
        pltpu.VMEM_SHARED((V + NS, D), jnp.float32),
        pltpu.SemaphoreType.DMA,
        pltpu.SemaphoreType.DMA,
        pltpu.SemaphoreType.DMA,
    ],
)(_body)


@jax.jit
def kernel(flat_tokens, cu_seqlens, vocab_emb):
    out = _tokenize(flat_tokens, cu_seqlens.astype(jnp.int32), vocab_emb)
    return out.reshape(B, L, D)


# speedup vs baseline: 1.0995x; 1.0042x over previous
import functools

import jax
import jax.numpy as jnp
from jax import lax
from jax.experimental import pallas as pl
from jax.experimental.pallas import tpu as pltpu
from jax.experimental.pallas import tpu_sc as plsc

B = 16
L = 2048
TOTAL = 16384
V = 8192
D = 128
NULL_ID = 0
START_ID = 2
END_ID = 3

NC = 2
NS = 16
LANES = 16
NW = NC * NS
PW = (B * L) // NW
FETCH = PW + 8
CHUNK = 64
NCH = PW // CHUNK
NBUF = 4


def _body(flat_hbm, cu_hbm, vocab_hbm, out_hbm,
          cu_v, tok_v, ids_v, rows_v, null_v, nullids_v, null_chunk_v,
          vocab_sp, gsem, ssem, stsem):
    sid = lax.axis_index("s")
    wid = lax.axis_index("c") * NS + sid
    b = wid // 2
    j0 = (wid % 2) * PW

    vshard = V // NS
    stage = pltpu.async_copy(vocab_hbm.at[pl.ds(sid * vshard, vshard)],
                             vocab_sp.at[pl.ds(sid * vshard, vshard)], stsem)

    pltpu.sync_copy(vocab_hbm.at[pl.ds(0, 1)], null_v)
    pltpu.sync_copy(null_v, vocab_sp.at[pl.ds(V + sid, 1)])

    pltpu.sync_copy(cu_hbm.at[pl.ds(0, 16)], cu_v)
    bvec = jnp.full((LANES,), b, jnp.int32)
    start = plsc.load_gather(cu_v, [bvec])[0]
    nxt = plsc.load_gather(cu_v, [jnp.minimum(bvec + 1, 15)])[0]
    seglen = jnp.where(b == B - 1, jnp.int32(TOTAL), nxt) - start

    s = start + j0 - 1
    sa = pl.multiple_of(jnp.maximum(0, jnp.minimum(s & ~7, TOTAL - FETCH)), 8)
    pltpu.sync_copy(flat_hbm.at[pl.ds(sa, FETCH)], tok_v)

    iota = jnp.arange(LANES, dtype=jnp.int32)

    def id_chunk(c, carry):
        for u in range(CHUNK // LANES):
            j = j0 + c * CHUNK + u * LANES + iota
            g = jnp.clip(start + j - 1, 0, TOTAL - 1)
            tok = plsc.load_gather(tok_v, [g - sa])
            pid = jnp.where(
                j == 0, jnp.int32(START_ID),
                jnp.where(j <= seglen, tok,
                          jnp.where(j == seglen + 1, jnp.int32(END_ID),
                                    jnp.int32(V) + sid)))
            ids_v[c, pl.ds(u * LANES, LANES)] = pid
        return carry

    for u in range(CHUNK // LANES):
        nullids_v[pl.ds(u * LANES, LANES)] = jnp.full(
            (LANES,), jnp.int32(V)) + sid

    lax.fori_loop(0, NCH, id_chunk, 0)

    stage.wait()
    plsc.subcore_barrier()

    pltpu.async_copy(vocab_sp.at[nullids_v], null_chunk_v, gsem).wait()

    def needs_gather(c):
        return (j0 + c * CHUNK) <= seglen + 1

    base = wid * PW
    gd = [None] * NBUF
    sd = [None] * NBUF
    for c in range(min(NBUF, NCH)):
        gd[c] = pltpu.make_async_copy(
            vocab_sp.at[ids_v.at[c]], rows_v.at[c], gsem)
        pl.when(needs_gather(c))(gd[c].start)
    for c in range(NCH):
        p = c % NBUF
        pl.when(needs_gather(c))(gd[p].wait)
        out_slc = out_hbm.at[pl.ds(base + c * CHUNK, CHUNK)]
        sd[p] = pltpu.make_async_copy(rows_v.at[p], out_slc, ssem)
        sd_null = pltpu.make_async_copy(null_chunk_v, out_slc, ssem)
        pl.when(needs_gather(c))(sd[p].start)
        pl.when(jnp.logical_not(needs_gather(c)))(sd_null.start)
        nc = c + NBUF
        if nc < NCH:
            sd[p].wait()
            gd[p] = pltpu.make_async_copy(
                vocab_sp.at[ids_v.at[nc]], rows_v.at[p], gsem)
            pl.when(needs_gather(nc))(gd[p].start)
    for c in range(NCH - NBUF, NCH):
        sd[c % NBUF].wait()


_tokenize = functools.partial(
    pl.kernel,
    out_type=jax.ShapeDtypeStruct((B * L, D), jnp.float32),
    mesh=plsc.VectorSubcoreMesh(core_axis_name="c", subcore_axis_name="s"),
    compiler_params=pltpu.CompilerParams(needs_layout_passes=False),
    scratch_types=[
        pltpu.VMEM((16,), jnp.int32),
        pltpu.VMEM((FETCH,), jnp.int32),
        pltpu.VMEM((NCH, CHUNK), jnp.int32),
        pltpu.VMEM((NBUF, CHUNK, D), jnp.float32),
        pltpu.VMEM((1, D), jnp.float32),
        pltpu.VMEM((CHUNK,), jnp.int32),
        pltpu.VMEM((CHUNK, D), jnp.float32),
        pltpu.VMEM_SHARED((V + NS, D), jnp.float32),
        pltpu.SemaphoreType.DMA,
        pltpu.SemaphoreType.DMA,
        pltpu.SemaphoreType.DMA,
    ],
)(_body)


@jax.jit
def kernel(flat_tokens, cu_seqlens, vocab_emb):
    out = _tokenize(flat_tokens, cu_seqlens.astype(jnp.int32), vocab_emb)
    return out.reshape(B, L, D)
